# Initial kernel scaffold; baseline (speedup 1.0000x reference)
import jax
import jax.numpy as jnp
from jax.experimental import pallas as pl


def kernel(density_grid, indices, sigmas):
    raise NotImplementedError("write your pallas kernel here")


# speedup vs baseline: 1.6382x; 1.6382x over previous
import functools

import jax
import jax.numpy as jnp
from jax import lax
from jax.experimental import pallas as pl
from jax.experimental.pallas import tpu as pltpu
from jax.experimental.pallas import tpu_sc as plsc

_CASCADE = 2
_CELLS = 128 ** 3
_DECAY = 0.95
_DTHRESH = 10.0
_NW = 32
_R = _CELLS // _NW
_L = 16
_ICHUNK = 8192
_NICHUNK = _CELLS // _ICHUNK
_ROW = 128
_NROW = _R // _ROW
_DCHUNK = 4096
_ROWS_PER_D = _DCHUNK // _ROW


def _sc_body(idx_hbm, sig_hbm, den_hbm, upd_hbm, part_hbm,
             win, idx_buf0, idx_buf1, gidx0, gidx1, gbuf0, gbuf1,
             d_buf, u_buf, p_buf,
             sem_i0, sem_i1, sem_g0, sem_g1):
  wid = lax.axis_index("s") * 2 + lax.axis_index("c")
  base = wid * _R
  iota = lax.iota(jnp.int32, _L)

  neg1 = jnp.full((_L,), -1, jnp.int32)

  def _init(t):
    win[pl.ds(t * _L, _L)] = neg1
  pl.loop(0, _R // _L)(_init)

  idx_bufs = (idx_buf0, idx_buf1)
  sems_i = (sem_i0, sem_i1)

  def _issue(c, p):
    pltpu.async_copy(idx_hbm.at[pl.ds(c * _ICHUNK, _ICHUNK)], idx_bufs[p],
                     sems_i[p])

  _issue(0, 0)

  def _chunk(c):
    for p in range(2):
      cc = c * 2 + p

      @pl.when(cc + 1 < _NICHUNK)
      def _():
        _issue(cc + 1, 1 - p)

      pltpu.make_async_copy(idx_hbm.at[pl.ds(0, _ICHUNK)], idx_bufs[p],
                            sems_i[p]).wait()
      ibase = cc * _ICHUNK

      def _vec(k):
        v = idx_bufs[p][pl.ds(k * _L, _L)]
        own = (v >> 16) == wid
        rel = v & 0xFFFF
        _, last = plsc.scan_count(v, mask=own)
        ival = (ibase + k * _L) + iota
        plsc.store_scatter(win, [rel], ival, mask=last & own)
      pl.loop(0, _ICHUNK // _L, unroll=4)(_vec)
  pl.loop(0, _NICHUNK // 2)(_chunk)

  gidxs = (gidx0, gidx1)
  gbufs = (gbuf0, gbuf1)
  sems_g = (sem_g0, sem_g1)
  zeros = jnp.zeros((_L,), jnp.float32)

  def _stage(j, cas, p):
    soff = jnp.int32(cas * _CELLS)

    def _st(k):
      v = win[pl.ds(j * _ROW + k * _L, _L)]
      gidxs[p][pl.ds(k * _L, _L)] = jnp.maximum(v, 0) + soff
    pl.loop(0, _ROW // _L, unroll=8)(_st)
    pltpu.async_copy(sig_hbm.at[gidxs[p]], gbufs[p], sems_g[p])

  def _cascade(cas, accs):
    _stage(0, cas, 0)

    def _rows(jj, acc):
      acc_s, acc_c = acc
      for p in range(2):
        j = jj * 2 + p

        @pl.when(j % _ROWS_PER_D == 0)
        def _():
          pltpu.sync_copy(
              den_hbm.at[cas, pl.ds(base + j * _ROW, _DCHUNK)], d_buf)

        @pl.when(j + 1 < _NROW)
        def _():
          _stage(j + 1, cas, 1 - p)

        pltpu.make_async_copy(sig_hbm.at[gidxs[p]], gbufs[p],
                              sems_g[p]).wait()
        dbase = (j % _ROWS_PER_D) * _ROW

        for k in range(_ROW // _L):
          v = win[pl.ds(j * _ROW + k * _L, _L)]
          hit = v >= 0
          g = gbufs[p][pl.ds(k * _L, _L)]
          tmp = jnp.where(hit, g, jnp.float32(-1.0))
          d = d_buf[pl.ds(dbase + k * _L, _L)]
          valid = d >= 0
          upd = jnp.where(valid, jnp.maximum(d * jnp.float32(_DECAY), tmp), d)
          u_buf[pl.ds(dbase + k * _L, _L)] = upd
          acc_s = acc_s + jnp.where(valid, upd, zeros)
          acc_c = acc_c + jnp.where(valid, jnp.float32(1.0), zeros)

        @pl.when(j % _ROWS_PER_D == _ROWS_PER_D - 1)
        def _():
          pltpu.sync_copy(
              u_buf,
              upd_hbm.at[cas, pl.ds(base + (j + 1) * _ROW - _DCHUNK,
                                    _DCHUNK)])
      return (acc_s, acc_c)
    return pl.loop(0, _NROW // 2, init_carry=accs)(_rows)

  accs = (zeros, zeros)
  for cas in range(_CASCADE):
    accs = _cascade(cas, accs)

  p_buf[pl.ds(0, _L)] = accs[0]
  p_buf[pl.ds(_L, _L)] = accs[1]
  pltpu.sync_copy(p_buf, part_hbm.at[wid])


def _sc_scatter_update(indices, sig_flat, density):
  kernel_fn = pl.kernel(
      _sc_body,
      out_type=[
          jax.ShapeDtypeStruct((_CASCADE, _CELLS), jnp.float32),
          jax.ShapeDtypeStruct((_NW, 2 * _L), jnp.float32),
      ],
      mesh=plsc.VectorSubcoreMesh(core_axis_name="c", subcore_axis_name="s"),
      compiler_params=pltpu.CompilerParams(needs_layout_passes=False),
      scratch_types=[
          pltpu.VMEM((_R,), jnp.int32),
          pltpu.VMEM((_ICHUNK,), jnp.int32),
          pltpu.VMEM((_ICHUNK,), jnp.int32),
          pltpu.VMEM((_ROW,), jnp.int32),
          pltpu.VMEM((_ROW,), jnp.int32),
          pltpu.VMEM((_ROW,), jnp.float32),
          pltpu.VMEM((_ROW,), jnp.float32),
          pltpu.VMEM((_DCHUNK,), jnp.float32),
          pltpu.VMEM((_DCHUNK,), jnp.float32),
          pltpu.VMEM((2 * _L,), jnp.float32),
          pltpu.SemaphoreType.DMA,
          pltpu.SemaphoreType.DMA,
          pltpu.SemaphoreType.DMA,
          pltpu.SemaphoreType.DMA,
      ],
  )
  return kernel_fn(indices, sig_flat, density)


def _pack_body(t_ref, x_ref, o_ref):
  t = t_ref[0, 0]
  occ = (x_ref[...] > t).astype(jnp.float32)
  li = lax.broadcasted_iota(jnp.int32, (128, 16), 0)
  ki = lax.broadcasted_iota(jnp.int32, (128, 16), 1)
  pmat = jnp.where(li // 8 == ki, jnp.int32(1) << (li % 8), 0)
  o_ref[...] = jnp.dot(occ, pmat.astype(jnp.float32),
                       preferred_element_type=jnp.float32)


def _tc_pack(updated, thresh):
  rows = _CASCADE * _CELLS // 128
  blk = 1024
  x = updated.reshape(rows, 128)
  out = pl.pallas_call(
      _pack_body,
      grid=(rows // blk,),
      in_specs=[
          pl.BlockSpec((1, 1), lambda i: (0, 0),
                       memory_space=pltpu.SMEM),
          pl.BlockSpec((blk, 128), lambda i: (i, 0)),
      ],
      out_specs=pl.BlockSpec((blk, 16), lambda i: (i, 0)),
      out_shape=jax.ShapeDtypeStruct((rows, 16), jnp.float32),
  )(thresh.reshape(1, 1), x)
  return out.astype(jnp.uint8).reshape(_CASCADE * _CELLS // 8)


def kernel(density_grid, indices, sigmas):
  sig_flat = sigmas.reshape(-1)
  updated, partials = _sc_scatter_update(indices, sig_flat, density_grid)
  total = jnp.sum(partials[:, :_L])
  count = jnp.sum(partials[:, _L:])
  mean_density = total / jnp.maximum(count, 1.0)
  thresh = jnp.minimum(mean_density, jnp.float32(_DTHRESH))
  bitfield = _tc_pack(updated, thresh)
  return updated, mean_density, bitfield


# speedup vs baseline: 1.8677x; 1.1401x over previous
import functools

import jax
import jax.numpy as jnp
from jax import lax
from jax.experimental import pallas as pl
from jax.experimental.pallas import tpu as pltpu
from jax.experimental.pallas import tpu_sc as plsc

_CASCADE = 2
_CELLS = 128 ** 3
_DECAY = 0.95
_DTHRESH = 10.0
_NW = 32
_R = _CELLS // _NW
_L = 16
_ICHUNK = 8192
_NICHUNK = _CELLS // _ICHUNK
_ROW = 128
_NROW = _R // _ROW
_DCHUNK = 4096
_ROWS_PER_D = _DCHUNK // _ROW


def _sc_body(idx_hbm, sig_hbm, den_hbm, upd_hbm, part_hbm,
             win, idx_buf0, idx_buf1, gidx0, gidx1, gbuf0, gbuf1,
             d_buf, u_buf, p_buf,
             sem_i0, sem_i1, sem_g0, sem_g1):
  wid = lax.axis_index("s") * 2 + lax.axis_index("c")
  base = wid * _R
  iota = lax.iota(jnp.int32, _L)

  neg1 = jnp.full((_L,), -1, jnp.int32)

  def _init(t):
    win[pl.ds(t * _L, _L)] = neg1
  pl.loop(0, _R // _L)(_init)

  idx_bufs = (idx_buf0, idx_buf1)
  sems_i = (sem_i0, sem_i1)

  def _issue(c, p):
    pltpu.async_copy(idx_hbm.at[pl.ds(c * _ICHUNK, _ICHUNK)], idx_bufs[p],
                     sems_i[p])

  _issue(0, 0)

  def _chunk(c):
    for p in range(2):
      cc = c * 2 + p

      @pl.when(cc + 1 < _NICHUNK)
      def _():
        _issue(cc + 1, 1 - p)

      pltpu.make_async_copy(idx_hbm.at[pl.ds(0, _ICHUNK)], idx_bufs[p],
                            sems_i[p]).wait()
      ibase = cc * _ICHUNK

      def _vec(k):
        v = idx_bufs[p][pl.ds(k * _L, _L)]
        own = (v >> 16) == wid
        rel = v & 0xFFFF
        ival = (ibase + k * _L) + iota
        plsc.store_scatter(win, [rel], ival, mask=own)
      pl.loop(0, _ICHUNK // _L, unroll=4)(_vec)
  pl.loop(0, _NICHUNK // 2)(_chunk)

  gidxs = (gidx0, gidx1)
  gbufs = (gbuf0, gbuf1)
  sems_g = (sem_g0, sem_g1)
  zeros = jnp.zeros((_L,), jnp.float32)

  def _stage(j, cas, p):
    soff = jnp.int32(cas * _CELLS)

    def _st(k):
      v = win[pl.ds(j * _ROW + k * _L, _L)]
      gidxs[p][pl.ds(k * _L, _L)] = jnp.maximum(v, 0) + soff
    pl.loop(0, _ROW // _L, unroll=8)(_st)
    pltpu.async_copy(sig_hbm.at[gidxs[p]], gbufs[p], sems_g[p])

  def _cascade(cas, accs):
    _stage(0, cas, 0)

    def _rows(jj, acc):
      acc_s, acc_c = acc
      for p in range(2):
        j = jj * 2 + p

        @pl.when(j % _ROWS_PER_D == 0)
        def _():
          pltpu.sync_copy(
              den_hbm.at[cas, pl.ds(base + j * _ROW, _DCHUNK)], d_buf)

        @pl.when(j + 1 < _NROW)
        def _():
          _stage(j + 1, cas, 1 - p)

        pltpu.make_async_copy(sig_hbm.at[gidxs[p]], gbufs[p],
                              sems_g[p]).wait()
        dbase = (j % _ROWS_PER_D) * _ROW

        for k in range(_ROW // _L):
          v = win[pl.ds(j * _ROW + k * _L, _L)]
          hit = v >= 0
          g = gbufs[p][pl.ds(k * _L, _L)]
          tmp = jnp.where(hit, g, jnp.float32(-1.0))
          d = d_buf[pl.ds(dbase + k * _L, _L)]
          valid = d >= 0
          upd = jnp.where(valid, jnp.maximum(d * jnp.float32(_DECAY), tmp), d)
          u_buf[pl.ds(dbase + k * _L, _L)] = upd
          acc_s = acc_s + jnp.where(valid, upd, zeros)
          acc_c = acc_c + jnp.where(valid, jnp.float32(1.0), zeros)

        @pl.when(j % _ROWS_PER_D == _ROWS_PER_D - 1)
        def _():
          pltpu.sync_copy(
              u_buf,
              upd_hbm.at[cas, pl.ds(base + (j + 1) * _ROW - _DCHUNK,
                                    _DCHUNK)])
      return (acc_s, acc_c)
    return pl.loop(0, _NROW // 2, init_carry=accs)(_rows)

  accs = (zeros, zeros)
  for cas in range(_CASCADE):
    accs = _cascade(cas, accs)

  p_buf[pl.ds(0, _L)] = accs[0]
  p_buf[pl.ds(_L, _L)] = accs[1]
  pltpu.sync_copy(p_buf, part_hbm.at[wid])


def _sc_scatter_update(indices, sig_flat, density):
  kernel_fn = pl.kernel(
      _sc_body,
      out_type=[
          jax.ShapeDtypeStruct((_CASCADE, _CELLS), jnp.float32),
          jax.ShapeDtypeStruct((_NW, 2 * _L), jnp.float32),
      ],
      mesh=plsc.VectorSubcoreMesh(core_axis_name="c", subcore_axis_name="s"),
      compiler_params=pltpu.CompilerParams(needs_layout_passes=False),
      scratch_types=[
          pltpu.VMEM((_R,), jnp.int32),
          pltpu.VMEM((_ICHUNK,), jnp.int32),
          pltpu.VMEM((_ICHUNK,), jnp.int32),
          pltpu.VMEM((_ROW,), jnp.int32),
          pltpu.VMEM((_ROW,), jnp.int32),
          pltpu.VMEM((_ROW,), jnp.float32),
          pltpu.VMEM((_ROW,), jnp.float32),
          pltpu.VMEM((_DCHUNK,), jnp.float32),
          pltpu.VMEM((_DCHUNK,), jnp.float32),
          pltpu.VMEM((2 * _L,), jnp.float32),
          pltpu.SemaphoreType.DMA,
          pltpu.SemaphoreType.DMA,
          pltpu.SemaphoreType.DMA,
          pltpu.SemaphoreType.DMA,
      ],
  )
  return kernel_fn(indices, sig_flat, density)


def _pack_body(t_ref, x_ref, o_ref):
  t = t_ref[0, 0]
  occ = (x_ref[...] > t).astype(jnp.float32)
  li = lax.broadcasted_iota(jnp.int32, (128, 16), 0)
  ki = lax.broadcasted_iota(jnp.int32, (128, 16), 1)
  pmat = jnp.where(li // 8 == ki, jnp.int32(1) << (li % 8), 0)
  o_ref[...] = jnp.dot(occ, pmat.astype(jnp.float32),
                       preferred_element_type=jnp.float32)


def _tc_pack(updated, thresh):
  rows = _CASCADE * _CELLS // 128
  blk = 1024
  x = updated.reshape(rows, 128)
  out = pl.pallas_call(
      _pack_body,
      grid=(rows // blk,),
      in_specs=[
          pl.BlockSpec((1, 1), lambda i: (0, 0),
                       memory_space=pltpu.SMEM),
          pl.BlockSpec((blk, 128), lambda i: (i, 0)),
      ],
      out_specs=pl.BlockSpec((blk, 16), lambda i: (i, 0)),
      out_shape=jax.ShapeDtypeStruct((rows, 16), jnp.float32),
  )(thresh.reshape(1, 1), x)
  return out.astype(jnp.uint8).reshape(_CASCADE * _CELLS // 8)


def kernel(density_grid, indices, sigmas):
  sig_flat = sigmas.reshape(-1)
  updated, partials = _sc_scatter_update(indices, sig_flat, density_grid)
  total = jnp.sum(partials[:, :_L])
  count = jnp.sum(partials[:, _L:])
  mean_density = total / jnp.maximum(count, 1.0)
  thresh = jnp.minimum(mean_density, jnp.float32(_DTHRESH))
  bitfield = _tc_pack(updated, thresh)
  return updated, mean_density, bitfield


# speedup vs baseline: 6.7661x; 3.6227x over previous
import functools

import jax
import jax.numpy as jnp
from jax import lax
from jax.experimental import pallas as pl
from jax.experimental.pallas import tpu as pltpu
from jax.experimental.pallas import tpu_sc as plsc

_CASCADE = 2
_CELLS = 128 ** 3
_DECAY = 0.95
_DTHRESH = 10.0
_NW = 32
_L = 16
_NR = 64
_R = _CELLS // _NR
_ICHUNK = 8192
_NICHUNK = _CELLS // _ICHUNK
_DCHUNK = 4096


def _sc_body(idx_hbm, sig_hbm, den_hbm, upd_hbm, part_hbm,
             tmp0, tmp1, ib0, ib1, s0b0, s0b1, s1b0, s1b1,
             d_buf, u_buf, p_buf, sem0, sem1):
  wid = lax.axis_index("s") * 2 + lax.axis_index("c")
  zeros = jnp.zeros((_L,), jnp.float32)
  neg1 = jnp.full((_L,), -1.0, jnp.float32)
  tmps = (tmp0, tmp1)
  ibs = (ib0, ib1)
  s0bs = (s0b0, s0b1)
  s1bs = (s1b0, s1b1)
  sems = (sem0, sem1)

  def _issue(c, p):
    off = c * _ICHUNK
    pltpu.async_copy(idx_hbm.at[pl.ds(off, _ICHUNK)], ibs[p], sems[p])
    pltpu.async_copy(sig_hbm.at[0, pl.ds(off, _ICHUNK)], s0bs[p], sems[p])
    pltpu.async_copy(sig_hbm.at[1, pl.ds(off, _ICHUNK)], s1bs[p], sems[p])

  def _drain(p):
    pltpu.make_async_copy(idx_hbm.at[pl.ds(0, _ICHUNK)], ibs[p],
                          sems[p]).wait()
    pltpu.make_async_copy(sig_hbm.at[0, pl.ds(0, _ICHUNK)], s0bs[p],
                          sems[p]).wait()
    pltpu.make_async_copy(sig_hbm.at[1, pl.ds(0, _ICHUNK)], s1bs[p],
                          sems[p]).wait()

  accs = (zeros, zeros)
  for half in range(2):
    wid2 = wid * 2 + half
    base = wid2 * _R

    def _init(t):
      tmp0[pl.ds(t * _L, _L)] = neg1
      tmp1[pl.ds(t * _L, _L)] = neg1
    pl.loop(0, _R // _L)(_init)

    _issue(0, 0)

    def _chunk(c):
      for p in range(2):
        cc = c * 2 + p

        @pl.when(cc + 1 < _NICHUNK)
        def _():
          _issue(cc + 1, 1 - p)

        _drain(p)

        def _vec(k):
          v = ibs[p][pl.ds(k * _L, _L)]
          own = (v >> 15) == wid2
          rel = v & 0x7FFF
          s0 = s0bs[p][pl.ds(k * _L, _L)]
          s1 = s1bs[p][pl.ds(k * _L, _L)]
          plsc.store_scatter(tmp0, [rel], s0, mask=own)
          plsc.store_scatter(tmp1, [rel], s1, mask=own)
        pl.loop(0, _ICHUNK // _L, unroll=4)(_vec)
    pl.loop(0, _NICHUNK // 2)(_chunk)

    for cas in range(_CASCADE):
      tmp = tmps[cas]

      def _blk(b, acc):
        pltpu.sync_copy(
            den_hbm.at[cas, pl.ds(base + b * _DCHUNK, _DCHUNK)], d_buf)

        def _cell(k, a):
          a_s, a_c = a
          d = d_buf[pl.ds(k * _L, _L)]
          t = tmp[pl.ds(b * _DCHUNK + k * _L, _L)]
          valid = d >= 0
          upd = jnp.where(valid, jnp.maximum(d * jnp.float32(_DECAY), t), d)
          u_buf[pl.ds(k * _L, _L)] = upd
          return (a_s + jnp.where(valid, upd, zeros),
                  a_c + jnp.where(valid, jnp.float32(1.0), zeros))
        acc = pl.loop(0, _DCHUNK // _L, init_carry=acc, unroll=4)(_cell)
        pltpu.sync_copy(
            u_buf, upd_hbm.at[cas, pl.ds(base + b * _DCHUNK, _DCHUNK)])
        return acc
      accs = pl.loop(0, _R // _DCHUNK, init_carry=accs)(_blk)

  p_buf[pl.ds(0, _L)] = accs[0]
  p_buf[pl.ds(_L, _L)] = accs[1]
  pltpu.sync_copy(p_buf, part_hbm.at[wid])


def _sc_scatter_update(indices, sigmas, density):
  kernel_fn = pl.kernel(
      _sc_body,
      out_type=[
          jax.ShapeDtypeStruct((_CASCADE, _CELLS), jnp.float32),
          jax.ShapeDtypeStruct((_NW, 2 * _L), jnp.float32),
      ],
      mesh=plsc.VectorSubcoreMesh(core_axis_name="c", subcore_axis_name="s"),
      compiler_params=pltpu.CompilerParams(needs_layout_passes=False),
      scratch_types=[
          pltpu.VMEM((_R,), jnp.float32),
          pltpu.VMEM((_R,), jnp.float32),
          pltpu.VMEM((_ICHUNK,), jnp.int32),
          pltpu.VMEM((_ICHUNK,), jnp.int32),
          pltpu.VMEM((_ICHUNK,), jnp.float32),
          pltpu.VMEM((_ICHUNK,), jnp.float32),
          pltpu.VMEM((_ICHUNK,), jnp.float32),
          pltpu.VMEM((_ICHUNK,), jnp.float32),
          pltpu.VMEM((_DCHUNK,), jnp.float32),
          pltpu.VMEM((_DCHUNK,), jnp.float32),
          pltpu.VMEM((2 * _L,), jnp.float32),
          pltpu.SemaphoreType.DMA,
          pltpu.SemaphoreType.DMA,
      ],
  )
  return kernel_fn(indices, sigmas, density)


def _pack_body(t_ref, x_ref, o_ref):
  t = t_ref[0, 0]
  occ = (x_ref[...] > t).astype(jnp.float32)
  li = lax.broadcasted_iota(jnp.int32, (128, 16), 0)
  ki = lax.broadcasted_iota(jnp.int32, (128, 16), 1)
  pmat = jnp.where(li // 8 == ki, jnp.int32(1) << (li % 8), 0)
  o_ref[...] = jnp.dot(occ, pmat.astype(jnp.float32),
                       preferred_element_type=jnp.float32)


def _tc_pack(updated, thresh):
  rows = _CASCADE * _CELLS // 128
  blk = 1024
  x = updated.reshape(rows, 128)
  out = pl.pallas_call(
      _pack_body,
      grid=(rows // blk,),
      in_specs=[
          pl.BlockSpec((1, 1), lambda i: (0, 0),
                       memory_space=pltpu.SMEM),
          pl.BlockSpec((blk, 128), lambda i: (i, 0)),
      ],
      out_specs=pl.BlockSpec((blk, 16), lambda i: (i, 0)),
      out_shape=jax.ShapeDtypeStruct((rows, 16), jnp.float32),
  )(thresh.reshape(1, 1), x)
  return out.astype(jnp.uint8).reshape(_CASCADE * _CELLS // 8)


def kernel(density_grid, indices, sigmas):
  updated, partials = _sc_scatter_update(indices, sigmas, density_grid)
  total = jnp.sum(partials[:, :_L])
  count = jnp.sum(partials[:, _L:])
  mean_density = total / jnp.maximum(count, 1.0)
  thresh = jnp.minimum(mean_density, jnp.float32(_DTHRESH))
  bitfield = _tc_pack(updated, thresh)
  return updated, mean_density, bitfield


# speedup vs baseline: 7.1759x; 1.0606x over previous
import functools

import jax
import jax.numpy as jnp
from jax import lax
from jax.experimental import pallas as pl
from jax.experimental.pallas import tpu as pltpu
from jax.experimental.pallas import tpu_sc as plsc

_CASCADE = 2
_CELLS = 128 ** 3
_DECAY = 0.95
_DTHRESH = 10.0
_NW = 32
_L = 16
_R = _CELLS // _NW
_ICHUNK = 8192
_NICHUNK = _CELLS // _ICHUNK
_DCHUNK = 4096


def _sc_body(idx_hbm, sig_hbm, den_hbm, upd_hbm, part_hbm,
             tmp, ib0, ib1, sb0, sb1,
             d_buf, u_buf, p_buf, sem0, sem1):
  wid = lax.axis_index("s") * 2 + lax.axis_index("c")
  base = wid * _R
  zeros = jnp.zeros((_L,), jnp.float32)
  neg1 = jnp.full((_L,), -1.0, jnp.float32)
  ibs = (ib0, ib1)
  sbs = (sb0, sb1)
  sems = (sem0, sem1)

  def _issue(cas, c, p):
    off = c * _ICHUNK
    pltpu.async_copy(idx_hbm.at[pl.ds(off, _ICHUNK)], ibs[p], sems[p])
    pltpu.async_copy(sig_hbm.at[cas, pl.ds(off, _ICHUNK)], sbs[p], sems[p])

  def _drain(p):
    pltpu.make_async_copy(idx_hbm.at[pl.ds(0, _ICHUNK)], ibs[p],
                          sems[p]).wait()
    pltpu.make_async_copy(sig_hbm.at[0, pl.ds(0, _ICHUNK)], sbs[p],
                          sems[p]).wait()

  accs = (zeros, zeros)
  for cas in range(_CASCADE):
    def _init(t):
      tmp[pl.ds(t * _L, _L)] = neg1
    pl.loop(0, _R // _L)(_init)

    _issue(cas, 0, 0)

    def _chunk(c):
      for p in range(2):
        cc = c * 2 + p

        @pl.when(cc + 1 < _NICHUNK)
        def _():
          _issue(cas, cc + 1, 1 - p)

        _drain(p)

        def _vec(k):
          v = ibs[p][pl.ds(k * _L, _L)]
          own = (v >> 16) == wid
          rel = v & 0xFFFF
          s = sbs[p][pl.ds(k * _L, _L)]
          plsc.store_scatter(tmp, [rel], s, mask=own)
        pl.loop(0, _ICHUNK // _L, unroll=8)(_vec)
    pl.loop(0, _NICHUNK // 2)(_chunk)

    def _blk(b, acc):
      pltpu.sync_copy(
          den_hbm.at[cas, pl.ds(base + b * _DCHUNK, _DCHUNK)], d_buf)

      def _cell(k, a):
        a_s, a_c = a
        d = d_buf[pl.ds(k * _L, _L)]
        t = tmp[pl.ds(b * _DCHUNK + k * _L, _L)]
        valid = d >= 0
        upd = jnp.where(valid, jnp.maximum(d * jnp.float32(_DECAY), t), d)
        u_buf[pl.ds(k * _L, _L)] = upd
        return (a_s + jnp.where(valid, upd, zeros),
                a_c + jnp.where(valid, jnp.float32(1.0), zeros))
      acc = pl.loop(0, _DCHUNK // _L, init_carry=acc, unroll=4)(_cell)
      pltpu.sync_copy(
          u_buf, upd_hbm.at[cas, pl.ds(base + b * _DCHUNK, _DCHUNK)])
      return acc
    accs = pl.loop(0, _R // _DCHUNK, init_carry=accs)(_blk)

  p_buf[pl.ds(0, _L)] = accs[0]
  p_buf[pl.ds(_L, _L)] = accs[1]
  pltpu.sync_copy(p_buf, part_hbm.at[wid])


def _sc_scatter_update(indices, sigmas, density):
  kernel_fn = pl.kernel(
      _sc_body,
      out_type=[
          jax.ShapeDtypeStruct((_CASCADE, _CELLS), jnp.float32),
          jax.ShapeDtypeStruct((_NW, 2 * _L), jnp.float32),
      ],
      mesh=plsc.VectorSubcoreMesh(core_axis_name="c", subcore_axis_name="s"),
      compiler_params=pltpu.CompilerParams(needs_layout_passes=False),
      scratch_types=[
          pltpu.VMEM((_R,), jnp.float32),
          pltpu.VMEM((_ICHUNK,), jnp.int32),
          pltpu.VMEM((_ICHUNK,), jnp.int32),
          pltpu.VMEM((_ICHUNK,), jnp.float32),
          pltpu.VMEM((_ICHUNK,), jnp.float32),
          pltpu.VMEM((_DCHUNK,), jnp.float32),
          pltpu.VMEM((_DCHUNK,), jnp.float32),
          pltpu.VMEM((2 * _L,), jnp.float32),
          pltpu.SemaphoreType.DMA,
          pltpu.SemaphoreType.DMA,
      ],
  )
  return kernel_fn(indices, sigmas, density)


def _pack_body(t_ref, x_ref, o_ref):
  t = t_ref[0, 0]
  occ = (x_ref[...] > t).astype(jnp.float32)
  li = lax.broadcasted_iota(jnp.int32, (128, 16), 0)
  ki = lax.broadcasted_iota(jnp.int32, (128, 16), 1)
  pmat = jnp.where(li // 8 == ki, jnp.int32(1) << (li % 8), 0)
  o_ref[...] = jnp.dot(occ, pmat.astype(jnp.float32),
                       preferred_element_type=jnp.float32)


def _tc_pack(updated, thresh):
  rows = _CASCADE * _CELLS // 128
  blk = 1024
  x = updated.reshape(rows, 128)
  out = pl.pallas_call(
      _pack_body,
      grid=(rows // blk,),
      in_specs=[
          pl.BlockSpec((1, 1), lambda i: (0, 0),
                       memory_space=pltpu.SMEM),
          pl.BlockSpec((blk, 128), lambda i: (i, 0)),
      ],
      out_specs=pl.BlockSpec((blk, 16), lambda i: (i, 0)),
      out_shape=jax.ShapeDtypeStruct((rows, 16), jnp.float32),
  )(thresh.reshape(1, 1), x)
  return out.astype(jnp.uint8).reshape(_CASCADE * _CELLS // 8)


def kernel(density_grid, indices, sigmas):
  updated, partials = _sc_scatter_update(indices, sigmas, density_grid)
  total = jnp.sum(partials[:, :_L])
  count = jnp.sum(partials[:, _L:])
  mean_density = total / jnp.maximum(count, 1.0)
  thresh = jnp.minimum(mean_density, jnp.float32(_DTHRESH))
  bitfield = _tc_pack(updated, thresh)
  return updated, mean_density, bitfield


# speedup vs baseline: 19.1640x; 2.6706x over previous
import jax
import jax.numpy as jnp
from jax import lax
from jax.experimental import pallas as pl
from jax.experimental.pallas import tpu as pltpu
from jax.experimental.pallas import tpu_sc as plsc

_CASCADE = 2
_CELLS = 128 ** 3
_DECAY = 0.95
_DTHRESH = 10.0
_NW = 32
_L = 16
_R = _CELLS // _NW
_ICHUNK = 8192
_NICHUNK = _CELLS // _ICHUNK
_DCHUNK = 4096


def _sc_body(idx_hbm, sig_hbm, den_hbm, upd_hbm, part_hbm,
             tmp, ib0, ib1, sb0, sb1,
             d_buf, u_buf, p_buf, sem0, sem1):
  wid = lax.axis_index("s") * 2 + lax.axis_index("c")
  base = wid * _R
  zeros = jnp.zeros((_L,), jnp.float32)
  neg1 = jnp.full((_L,), -1.0, jnp.float32)
  ibs = (ib0, ib1)
  sbs = (sb0, sb1)
  sems = (sem0, sem1)

  def _issue(cas, c, p):
    off = c * _ICHUNK
    pltpu.async_copy(idx_hbm.at[pl.ds(off, _ICHUNK)], ibs[p], sems[p])
    pltpu.async_copy(sig_hbm.at[cas, pl.ds(off, _ICHUNK)], sbs[p], sems[p])

  def _drain(p):
    pltpu.make_async_copy(idx_hbm.at[pl.ds(0, _ICHUNK)], ibs[p],
                          sems[p]).wait()
    pltpu.make_async_copy(sig_hbm.at[0, pl.ds(0, _ICHUNK)], sbs[p],
                          sems[p]).wait()

  accs = (zeros, zeros)
  for cas in range(_CASCADE):
    def _init(t):
      tmp[pl.ds(t * _L, _L)] = neg1
    pl.loop(0, _R // _L)(_init)

    _issue(cas, 0, 0)

    def _chunk(c):
      for p in range(2):
        cc = c * 2 + p

        @pl.when(cc + 1 < _NICHUNK)
        def _():
          _issue(cas, cc + 1, 1 - p)

        _drain(p)

        def _vec(k):
          vs = [ibs[p][pl.ds((k * 8 + j) * _L, _L)] for j in range(8)]
          ss = [sbs[p][pl.ds((k * 8 + j) * _L, _L)] for j in range(8)]
          for j in range(8):
            own = (vs[j] >> 16) == wid
            rel = vs[j] & 0xFFFF
            plsc.store_scatter(tmp, [rel], ss[j], mask=own)
        pl.loop(0, _ICHUNK // _L // 8)(_vec)
    pl.loop(0, _NICHUNK // 2)(_chunk)

    def _blk(b, acc):
      pltpu.sync_copy(
          den_hbm.at[cas, pl.ds(base + b * _DCHUNK, _DCHUNK)], d_buf)

      def _cell(k, a):
        a_s, a_c = a
        d = d_buf[pl.ds(k * _L, _L)]
        t = tmp[pl.ds(b * _DCHUNK + k * _L, _L)]
        valid = d >= 0
        upd = jnp.where(valid, jnp.maximum(d * jnp.float32(_DECAY), t), d)
        u_buf[pl.ds(k * _L, _L)] = upd
        return (a_s + jnp.where(valid, upd, zeros),
                a_c + jnp.where(valid, jnp.float32(1.0), zeros))
      acc = pl.loop(0, _DCHUNK // _L, init_carry=acc, unroll=4)(_cell)
      pltpu.sync_copy(
          u_buf, upd_hbm.at[cas, pl.ds(base + b * _DCHUNK, _DCHUNK)])
      return acc
    accs = pl.loop(0, _R // _DCHUNK, init_carry=accs)(_blk)

  p_buf[pl.ds(0, _L)] = accs[0]
  p_buf[pl.ds(_L, _L)] = accs[1]
  pltpu.sync_copy(p_buf, part_hbm.at[wid])


def _sc_scatter_update(indices, sigmas, density):
  kernel_fn = pl.kernel(
      _sc_body,
      out_type=[
          jax.ShapeDtypeStruct((_CASCADE, _CELLS), jnp.float32),
          jax.ShapeDtypeStruct((_NW, 2 * _L), jnp.float32),
      ],
      mesh=plsc.VectorSubcoreMesh(core_axis_name="c", subcore_axis_name="s"),
      compiler_params=pltpu.CompilerParams(needs_layout_passes=False),
      scratch_types=[
          pltpu.VMEM((_R,), jnp.float32),
          pltpu.VMEM((_ICHUNK,), jnp.int32),
          pltpu.VMEM((_ICHUNK,), jnp.int32),
          pltpu.VMEM((_ICHUNK,), jnp.float32),
          pltpu.VMEM((_ICHUNK,), jnp.float32),
          pltpu.VMEM((_DCHUNK,), jnp.float32),
          pltpu.VMEM((_DCHUNK,), jnp.float32),
          pltpu.VMEM((2 * _L,), jnp.float32),
          pltpu.SemaphoreType.DMA,
          pltpu.SemaphoreType.DMA,
      ],
  )
  return kernel_fn(indices, sigmas, density)


def _pack_body(t_ref, x_ref, o_ref):
  t = t_ref[0, 0]
  occ = (x_ref[...] > t).astype(jnp.float32)
  li = lax.broadcasted_iota(jnp.int32, (128, 16), 0)
  ki = lax.broadcasted_iota(jnp.int32, (128, 16), 1)
  pmat = jnp.where(li // 8 == ki, jnp.int32(1) << (li % 8), 0)
  o_ref[...] = jnp.dot(occ, pmat.astype(jnp.float32),
                       preferred_element_type=jnp.float32)


def _tc_pack(updated, thresh):
  rows = _CASCADE * _CELLS // 128
  blk = 1024
  x = updated.reshape(rows, 128)
  out = pl.pallas_call(
      _pack_body,
      grid=(rows // blk,),
      in_specs=[
          pl.BlockSpec((1, 1), lambda i: (0, 0),
                       memory_space=pltpu.SMEM),
          pl.BlockSpec((blk, 128), lambda i: (i, 0)),
      ],
      out_specs=pl.BlockSpec((blk, 16), lambda i: (i, 0)),
      out_shape=jax.ShapeDtypeStruct((rows, 16), jnp.float32),
  )(thresh.reshape(1, 1), x)
  return out.astype(jnp.uint8).reshape(_CASCADE * _CELLS // 8)


def kernel(density_grid, indices, sigmas):
  updated, partials = _sc_scatter_update(indices, sigmas, density_grid)
  total = jnp.sum(partials[:, :_L])
  count = jnp.sum(partials[:, _L:])
  mean_density = total / jnp.maximum(count, 1.0)
  thresh = jnp.minimum(mean_density, jnp.float32(_DTHRESH))
  bitfield = _tc_pack(updated, thresh)
  return updated, mean_density, bitfield
